# initial kernel scaffold (unmeasured)
import jax
import jax.numpy as jnp
from jax import lax
from jax.experimental import pallas as pl
from jax.experimental.pallas import tpu as pltpu


def kernel(Q, K, V):
    b, s, h, d = Q.shape
    scale = d ** -0.5

    def body(q_ref, k_ref, v_ref, k_any, v_any, out_ref,
             krem, vrem, ksend, krecv, vsend, vrecv):
        hh = pl.program_id(0)
        my_x = lax.axis_index("x")
        my_y = lax.axis_index("y")
        my_z = lax.axis_index("z")
        nbr = (1 - my_x, my_y, my_z)

        def descriptors(i):
            k_copy = pltpu.make_async_remote_copy(
                src_ref=k_any.at[0, :, i, :],
                dst_ref=krem.at[i],
                send_sem=ksend.at[i],
                recv_sem=krecv.at[i],
                device_id=nbr,
                device_id_type=pl.DeviceIdType.MESH,
            )
            v_copy = pltpu.make_async_remote_copy(
                src_ref=v_any.at[0, :, i, :],
                dst_ref=vrem.at[i],
                send_sem=vsend.at[i],
                recv_sem=vrecv.at[i],
                device_id=nbr,
                device_id_type=pl.DeviceIdType.MESH,
            )
            return k_copy, v_copy

        @pl.when(hh == 0)
        def _():
            barrier = pltpu.get_barrier_semaphore()
            pl.semaphore_signal(barrier, inc=1, device_id=nbr,
                                device_id_type=pl.DeviceIdType.MESH)
            pl.semaphore_wait(barrier, 1)
            for i in range(h):
                kc, vc = descriptors(i)
                kc.start()
                vc.start()

        q = q_ref[0, :, 0, :]
        k1 = k_ref[0, :, 0, :]
        v1 = v_ref[0, :, 0, :]
        s1 = lax.dot_general(q, k1, (((1,), (1,)), ((), ())),
                             preferred_element_type=jnp.float32) * scale
        m1 = jnp.max(s1, axis=1, keepdims=True)
        p1 = jnp.exp(s1 - m1)
        l1 = jnp.sum(p1, axis=1, keepdims=True)
        o1 = lax.dot_general(p1, v1, (((1,), (0,)), ((), ())),
                             preferred_element_type=jnp.float32)

        kc, vc = descriptors(hh)
        kc.wait_recv()
        vc.wait_recv()
        k2 = krem[hh]
        v2 = vrem[hh]
        s2 = lax.dot_general(q, k2, (((1,), (1,)), ((), ())),
                             preferred_element_type=jnp.float32) * scale
        m2 = jnp.max(s2, axis=1, keepdims=True)
        p2 = jnp.exp(s2 - m2)
        l2 = jnp.sum(p2, axis=1, keepdims=True)
        o2 = lax.dot_general(p2, v2, (((1,), (0,)), ((), ())),
                             preferred_element_type=jnp.float32)

        m = jnp.maximum(m1, m2)
        a1 = jnp.exp(m1 - m)
        a2 = jnp.exp(m2 - m)
        out_ref[0, :, 0, :] = (o1 * a1 + o2 * a2) / (l1 * a1 + l2 * a2)

        @pl.when(hh == h - 1)
        def _():
            for i in range(h):
                kc, vc = descriptors(i)
                kc.wait_send()
                vc.wait_send()

    blk = pl.BlockSpec((1, s, 1, d), lambda i: (0, 0, i, 0))
    return pl.pallas_call(
        body,
        grid=(h,),
        out_shape=jax.ShapeDtypeStruct((b, s, h, d), jnp.float32),
        in_specs=[
            blk,
            blk,
            blk,
            pl.BlockSpec(memory_space=pltpu.ANY),
            pl.BlockSpec(memory_space=pltpu.ANY),
        ],
        out_specs=blk,
        scratch_shapes=[
            pltpu.VMEM((h, s, d), jnp.float32),
            pltpu.VMEM((h, s, d), jnp.float32),
            pltpu.SemaphoreType.DMA((h,)),
            pltpu.SemaphoreType.DMA((h,)),
            pltpu.SemaphoreType.DMA((h,)),
            pltpu.SemaphoreType.DMA((h,)),
        ],
        compiler_params=pltpu.CompilerParams(
            collective_id=0,
            dimension_semantics=("arbitrary",),
        ),
    )(Q, K, V, K, V)


# baseline (device time: 204872 ns/iter reference)
import jax
import jax.numpy as jnp
from jax import lax
from jax.experimental import pallas as pl
from jax.experimental.pallas import tpu as pltpu


def kernel(Q, K, V):
    b, s, h, d = Q.shape
    scale = d ** -0.5

    def body(q_ref, k_ref, v_ref, k_any, v_any, out_ref,
             krem, vrem, ksend, krecv, vsend, vrecv):
        hh = pl.program_id(0)
        my_x = lax.axis_index("x")
        my_y = lax.axis_index("y")
        my_z = lax.axis_index("z")
        nbr = (1 - my_x, my_y, my_z)

        def descriptors(i):
            k_copy = pltpu.make_async_remote_copy(
                src_ref=k_any.at[0, :, i, :],
                dst_ref=krem.at[i],
                send_sem=ksend.at[i],
                recv_sem=krecv.at[i],
                device_id=nbr,
                device_id_type=pl.DeviceIdType.MESH,
            )
            v_copy = pltpu.make_async_remote_copy(
                src_ref=v_any.at[0, :, i, :],
                dst_ref=vrem.at[i],
                send_sem=vsend.at[i],
                recv_sem=vrecv.at[i],
                device_id=nbr,
                device_id_type=pl.DeviceIdType.MESH,
            )
            return k_copy, v_copy

        @pl.when(hh == 0)
        def _():
            barrier = pltpu.get_barrier_semaphore()
            pl.semaphore_signal(barrier, inc=1, device_id=nbr,
                                device_id_type=pl.DeviceIdType.MESH)
            pl.semaphore_wait(barrier, 1)
            for i in range(h):
                kc, vc = descriptors(i)
                kc.start()
                vc.start()

        q = q_ref[0, :, hh, :]
        k1 = k_ref[0, :, hh, :]
        v1 = v_ref[0, :, hh, :]
        s1 = lax.dot_general(q, k1, (((1,), (1,)), ((), ())),
                             preferred_element_type=jnp.float32) * scale
        m1 = jnp.max(s1, axis=1, keepdims=True)
        p1 = jnp.exp(s1 - m1)
        l1 = jnp.sum(p1, axis=1, keepdims=True)
        o1 = lax.dot_general(p1, v1, (((1,), (0,)), ((), ())),
                             preferred_element_type=jnp.float32)

        kc, vc = descriptors(hh)
        kc.wait_recv()
        vc.wait_recv()
        k2 = krem[hh]
        v2 = vrem[hh]
        s2 = lax.dot_general(q, k2, (((1,), (1,)), ((), ())),
                             preferred_element_type=jnp.float32) * scale
        m2 = jnp.max(s2, axis=1, keepdims=True)
        p2 = jnp.exp(s2 - m2)
        l2 = jnp.sum(p2, axis=1, keepdims=True)
        o2 = lax.dot_general(p2, v2, (((1,), (0,)), ((), ())),
                             preferred_element_type=jnp.float32)

        m = jnp.maximum(m1, m2)
        a1 = jnp.exp(m1 - m)
        a2 = jnp.exp(m2 - m)
        out_ref[0, :, hh, :] = (o1 * a1 + o2 * a2) / (l1 * a1 + l2 * a2)

        @pl.when(hh == h - 1)
        def _():
            for i in range(h):
                kc, vc = descriptors(i)
                kc.wait_send()
                vc.wait_send()

    vmem = pl.BlockSpec(memory_space=pltpu.MemorySpace.VMEM)
    return pl.pallas_call(
        body,
        grid=(h,),
        out_shape=jax.ShapeDtypeStruct((b, s, h, d), jnp.float32),
        in_specs=[
            vmem,
            vmem,
            vmem,
            pl.BlockSpec(memory_space=pl.ANY),
            pl.BlockSpec(memory_space=pl.ANY),
        ],
        out_specs=vmem,
        scratch_shapes=[
            pltpu.VMEM((h, s, d), jnp.float32),
            pltpu.VMEM((h, s, d), jnp.float32),
            pltpu.SemaphoreType.DMA((h,)),
            pltpu.SemaphoreType.DMA((h,)),
            pltpu.SemaphoreType.DMA((h,)),
            pltpu.SemaphoreType.DMA((h,)),
        ],
        compiler_params=pltpu.CompilerParams(
            collective_id=0,
            dimension_semantics=("arbitrary",),
            vmem_limit_bytes=100 * 1024 * 1024,
        ),
    )(Q, K, V, K, V)
